# baseline (speedup 1.0000x reference)
import jax
import jax.numpy as jnp
from jax import lax
from jax.experimental import pallas as pl
from jax.experimental.pallas import tpu as pltpu

RELATION_THRESHOLD = 0.5
_BN = 2000
_G = 160
_R = 125


def _main_body(orig_ref, size_ref, scores_ref, sboxf_ref, oboxf_ref,
               scat_ref, ocat_ref, verb_ref, keep_ref, boxes_ref,
               labels_ref):
    b = pl.program_id(0)

    x = scores_ref[0]
    m = jnp.max(x, axis=-1, keepdims=True)
    e = jnp.exp(x - m)
    s_row = jnp.sum(e, axis=-1, keepdims=True)
    keep_row = (1.0 / s_row) >= RELATION_THRESHOLD
    verb_ref[0] = jnp.where(keep_row, x, 0.0)

    ones_c = jnp.ones((1, x.shape[-1]), jnp.float32)
    s_lane = lax.dot_general(ones_c, e, (((1,), (1,)), ((), ())),
                             preferred_element_type=jnp.float32)
    keep_lane = (1.0 / s_lane) >= RELATION_THRESHOLD
    keep_ref[0, 0, 0] = keep_lane.astype(jnp.int32)[0]

    sw = orig_ref[b, 1] / size_ref[b, 1]
    sh = orig_ref[b, 0] / size_ref[b, 0]
    lidx = lax.broadcasted_iota(jnp.int32, (1, 4 * _BN), 1)
    scale = jnp.where((lidx % 2) == 0, sw, sh)
    boxes_ref[0, 0, 0, 0] = (sboxf_ref[0, 0, 0] * scale)[0]
    boxes_ref[0, 1, 0, 0] = (oboxf_ref[0, 0, 0] * scale)[0]
    labels_ref[0, 0, 0, 0] = scat_ref[0, 0, 0] - 1
    labels_ref[0, 1, 0, 0] = ocat_ref[0, 0, 0] - 1


def _ids_body(triR_ref, triG_ref, keep_ref, subids_ref, objids_ref):
    k = keep_ref[0].astype(jnp.float32)
    pref = jnp.dot(k, triR_ref[...], preferred_element_type=jnp.float32)
    rowsum = pref[:, _R - 1:_R]
    offs = jnp.dot(triG_ref[...], rowsum,
                   preferred_element_type=jnp.float32)
    csum = pref + offs
    total = offs[_G - 1, 0] + rowsum[_G - 1, 0]
    rank = csum.astype(jnp.int32) - 1
    kept = k > 0.5
    sub = jnp.where(kept, rank, -1)
    obj = jnp.where(kept, rank + total.astype(jnp.int32), -1)
    subids_ref[0] = sub
    objids_ref[0] = obj


def kernel(relation_scores, subject_bbox, object_bbox, subject_category,
           object_category, orig_size, size):
    B, N, C = relation_scores.shape
    nb = N // _BN
    scat4 = subject_category.reshape(B, nb, 1, _BN)
    ocat4 = object_category.reshape(B, nb, 1, _BN)
    sboxf = subject_bbox.reshape(B, nb, 1, 4 * _BN)
    oboxf = object_bbox.reshape(B, nb, 1, 4 * _BN)

    verb, keep4, boxes5, labels5 = pl.pallas_call(
        _main_body,
        grid=(B, nb),
        in_specs=[
            pl.BlockSpec(memory_space=pltpu.SMEM),
            pl.BlockSpec(memory_space=pltpu.SMEM),
            pl.BlockSpec((1, _BN, C), lambda b, i: (b, i, 0)),
            pl.BlockSpec((1, 1, 1, 4 * _BN), lambda b, i: (b, i, 0, 0)),
            pl.BlockSpec((1, 1, 1, 4 * _BN), lambda b, i: (b, i, 0, 0)),
            pl.BlockSpec((1, 1, 1, _BN), lambda b, i: (b, i, 0, 0)),
            pl.BlockSpec((1, 1, 1, _BN), lambda b, i: (b, i, 0, 0)),
        ],
        out_specs=[
            pl.BlockSpec((1, _BN, C), lambda b, i: (b, i, 0)),
            pl.BlockSpec((1, 1, 1, _BN), lambda b, i: (b, i, 0, 0)),
            pl.BlockSpec((1, 2, 1, 1, 4 * _BN), lambda b, i: (b, 0, i, 0, 0)),
            pl.BlockSpec((1, 2, 1, 1, _BN), lambda b, i: (b, 0, i, 0, 0)),
        ],
        out_shape=[
            jax.ShapeDtypeStruct((B, N, C), jnp.float32),
            jax.ShapeDtypeStruct((B, nb, 1, _BN), jnp.int32),
            jax.ShapeDtypeStruct((B, 2, nb, 1, 4 * _BN), jnp.float32),
            jax.ShapeDtypeStruct((B, 2, nb, 1, _BN), jnp.int32),
        ],
        compiler_params=pltpu.CompilerParams(
            dimension_semantics=("arbitrary", "arbitrary")),
    )(orig_size, size, relation_scores, sboxf, oboxf, scat4, ocat4)

    row = lax.broadcasted_iota(jnp.int32, (_R, _R), 0)
    col = lax.broadcasted_iota(jnp.int32, (_R, _R), 1)
    triR = (row <= col).astype(jnp.float32)
    rowg = lax.broadcasted_iota(jnp.int32, (_G, _G), 0)
    colg = lax.broadcasted_iota(jnp.int32, (_G, _G), 1)
    triG = (colg < rowg).astype(jnp.float32)

    keep_gr = keep4.reshape(B, _G, _R)
    subids4, objids4 = pl.pallas_call(
        _ids_body,
        grid=(B,),
        in_specs=[
            pl.BlockSpec((_R, _R), lambda b: (0, 0)),
            pl.BlockSpec((_G, _G), lambda b: (0, 0)),
            pl.BlockSpec((1, _G, _R), lambda b: (b, 0, 0)),
        ],
        out_specs=[
            pl.BlockSpec((1, _G, _R), lambda b: (b, 0, 0)),
            pl.BlockSpec((1, _G, _R), lambda b: (b, 0, 0)),
        ],
        out_shape=[
            jax.ShapeDtypeStruct((B, _G, _R), jnp.int32),
            jax.ShapeDtypeStruct((B, _G, _R), jnp.int32),
        ],
        compiler_params=pltpu.CompilerParams(
            dimension_semantics=("arbitrary",)),
    )(triR, triG, keep_gr)

    boxes = boxes5.reshape(B, 2 * N, 4)
    labels = labels5.reshape(B, 2 * N)
    keep = keep4.reshape(B, N).astype(bool)
    sub_ids = subids4.reshape(B, N)
    obj_ids = objids4.reshape(B, N)
    return boxes, labels, verb, keep, sub_ids, obj_ids


# speedup vs baseline: 1.0664x; 1.0664x over previous
import jax
import jax.numpy as jnp
from jax import lax
from jax.experimental import pallas as pl
from jax.experimental.pallas import tpu as pltpu

RELATION_THRESHOLD = 0.5
_BN = 1000


def _main_body(orig_ref, size_ref, tri_ref, scores_ref, sbox_ref, obox_ref,
               scat_ref, ocat_ref, verb_ref, boxes_ref, labels_ref, keep_ref,
               subids_ref, nkeep_ref, cnt_ref):
    b = pl.program_id(0)
    i = pl.program_id(1)

    @pl.when(i == 0)
    def _init():
        cnt_ref[0] = 0

    x = scores_ref[0]
    m = jnp.max(x, axis=-1, keepdims=True)
    s = jnp.sum(jnp.exp(x - m), axis=-1, keepdims=True)
    keep2 = (1.0 / s) >= RELATION_THRESHOLD
    verb_ref[0] = jnp.where(keep2, x, 0.0)

    kf = keep2.astype(jnp.float32)
    csum = jnp.dot(tri_ref[...], kf, preferred_element_type=jnp.float32)
    base = cnt_ref[0]
    rank = base + csum.astype(jnp.int32) - 1
    ids = jnp.where(keep2, rank, -1)
    subids_ref[0, 0, 0] = ids.reshape(1, _BN)[0]
    keep_ref[0, 0, 0] = keep2.astype(jnp.int32).reshape(1, _BN)[0]
    total = base + jnp.sum(kf).astype(jnp.int32)
    cnt_ref[0] = total
    nkeep_ref[0, 0, 0] = total

    sw = orig_ref[b, 1] / size_ref[b, 1]
    sh = orig_ref[b, 0] / size_ref[b, 0]
    lidx = lax.broadcasted_iota(jnp.int32, (_BN, 4), 1)
    scale = jnp.where((lidx % 2) == 0, sw, sh)
    boxes_ref[0, 0] = sbox_ref[0] * scale
    boxes_ref[0, 1] = obox_ref[0] * scale
    labels_ref[0, 0, 0, 0] = scat_ref[0, 0, 0] - 1
    labels_ref[0, 1, 0, 0] = ocat_ref[0, 0, 0] - 1


def _objids_body(keep_ref, subids_ref, nkeep_ref, objids_ref):
    b = pl.program_id(0)
    nk = nkeep_ref[b, 0, 0]
    k = keep_ref[0, 0, 0]
    objids_ref[0, 0, 0] = jnp.where(k > 0, subids_ref[0, 0, 0] + nk, -1)


def kernel(relation_scores, subject_bbox, object_bbox, subject_category,
           object_category, orig_size, size):
    B, N, C = relation_scores.shape
    nb = N // _BN
    scat4 = subject_category.reshape(B, nb, 1, _BN)
    ocat4 = object_category.reshape(B, nb, 1, _BN)
    row = lax.broadcasted_iota(jnp.int32, (_BN, _BN), 0)
    col = lax.broadcasted_iota(jnp.int32, (_BN, _BN), 1)
    tri = (col <= row).astype(jnp.float32)

    grid = (B, nb)
    verb, boxes4, labels3, keep3, subids3, nkeep = pl.pallas_call(
        _main_body,
        grid=grid,
        in_specs=[
            pl.BlockSpec(memory_space=pltpu.SMEM),
            pl.BlockSpec(memory_space=pltpu.SMEM),
            pl.BlockSpec((_BN, _BN), lambda b, i: (0, 0)),
            pl.BlockSpec((1, _BN, C), lambda b, i: (b, i, 0)),
            pl.BlockSpec((1, _BN, 4), lambda b, i: (b, i, 0)),
            pl.BlockSpec((1, _BN, 4), lambda b, i: (b, i, 0)),
            pl.BlockSpec((1, 1, 1, _BN), lambda b, i: (b, i, 0, 0)),
            pl.BlockSpec((1, 1, 1, _BN), lambda b, i: (b, i, 0, 0)),
        ],
        out_specs=[
            pl.BlockSpec((1, _BN, C), lambda b, i: (b, i, 0)),
            pl.BlockSpec((1, 2, _BN, 4), lambda b, i: (b, 0, i, 0)),
            pl.BlockSpec((1, 2, 1, 1, _BN), lambda b, i: (b, 0, i, 0, 0)),
            pl.BlockSpec((1, 1, 1, _BN), lambda b, i: (b, i, 0, 0)),
            pl.BlockSpec((1, 1, 1, _BN), lambda b, i: (b, i, 0, 0)),
            pl.BlockSpec((1, 1, 1), lambda b, i: (b, 0, 0),
                         memory_space=pltpu.SMEM),
        ],
        out_shape=[
            jax.ShapeDtypeStruct((B, N, C), jnp.float32),
            jax.ShapeDtypeStruct((B, 2, N, 4), jnp.float32),
            jax.ShapeDtypeStruct((B, 2, nb, 1, _BN), jnp.int32),
            jax.ShapeDtypeStruct((B, nb, 1, _BN), jnp.int32),
            jax.ShapeDtypeStruct((B, nb, 1, _BN), jnp.int32),
            jax.ShapeDtypeStruct((B, 1, 1), jnp.int32),
        ],
        scratch_shapes=[pltpu.SMEM((1,), jnp.int32)],
        compiler_params=pltpu.CompilerParams(
            dimension_semantics=("arbitrary", "arbitrary")),
    )(orig_size, size, tri, relation_scores, subject_bbox, object_bbox,
      scat4, ocat4)

    objids3 = pl.pallas_call(
        _objids_body,
        grid=grid,
        in_specs=[
            pl.BlockSpec((1, 1, 1, _BN), lambda b, i: (b, i, 0, 0)),
            pl.BlockSpec((1, 1, 1, _BN), lambda b, i: (b, i, 0, 0)),
            pl.BlockSpec(memory_space=pltpu.SMEM),
        ],
        out_specs=pl.BlockSpec((1, 1, 1, _BN), lambda b, i: (b, i, 0, 0)),
        out_shape=jax.ShapeDtypeStruct((B, nb, 1, _BN), jnp.int32),
        compiler_params=pltpu.CompilerParams(
            dimension_semantics=("arbitrary", "arbitrary")),
    )(keep3, subids3, nkeep)

    boxes = boxes4.reshape(B, 2 * N, 4)
    labels = labels3.reshape(B, 2 * N)
    keep = keep3.reshape(B, N).astype(bool)
    sub_ids = subids3.reshape(B, N)
    obj_ids = objids3.reshape(B, N)
    return boxes, labels, verb, keep, sub_ids, obj_ids
